# baseline (speedup 1.0000x reference)
import functools

import jax
import jax.numpy as jnp
from jax import lax
from jax.experimental import pallas as pl
from jax.experimental.pallas import tpu as pltpu
from jax.experimental.pallas import tpu_sc as plsc

_B = 4096
_F = 26
_D = 16
_N = _B * _F
_NW = 32
_CHUNK = _N // _NW
_V = 1000000
_TAU = 0.1
_SIG_HALF = 1.0 / (1.0 + 2.718281828459045 ** (-0.5))

_TBLK = 8192


def _tr_body(wt_ref, out_ref):
    eye = jnp.eye(_D, dtype=jnp.float32)
    out_ref[...] = lax.dot_general(
        wt_ref[...], eye, (((0,), (0,)), ((), ())),
        preferred_element_type=jnp.float32)


def _transpose_tc(wt):
    grid = (_V + _TBLK - 1) // _TBLK
    return pl.pallas_call(
        _tr_body,
        grid=(grid,),
        in_specs=[pl.BlockSpec((_D, _TBLK), lambda j: (0, j))],
        out_specs=pl.BlockSpec((_TBLK, _D), lambda j: (j, 0)),
        out_shape=jax.ShapeDtypeStruct((_V, _D), jnp.float32),
    )(wt)


def _sc_body(x_hbm, w_hbm, m_hbm, out_hbm, idx_v, rows_v, mask_v, scale_v,
             sem_w, sem_m):
    wid = lax.axis_index("s") * 2 + lax.axis_index("c")
    base = wid * _CHUNK
    pltpu.sync_copy(x_hbm.at[pl.ds(base, _CHUNK)], idx_v)
    cw = pltpu.async_copy(w_hbm.at[idx_v], rows_v, sem_w)
    cm = pltpu.async_copy(m_hbm.at[idx_v], mask_v, sem_m)
    cm.wait()

    inv_tau = jnp.float32(1.0 / _TAU)
    scale_c = jnp.float32(1.0 / _SIG_HALF)

    def scale_body(g, carry):
        m = mask_v[pl.ds(g * 16, 16)]
        s = scale_c / (1.0 + jnp.exp(m * -inv_tau))
        scale_v[pl.ds(g * 16, 16)] = s
        return carry

    lax.fori_loop(0, _CHUNK // 16, scale_body, 0)
    cw.wait()

    def mul_body(g, carry):
        s = scale_v[pl.ds(g * 16, 16)]
        for j in range(16):
            rows_v[g * 16 + j, :] = rows_v[g * 16 + j, :] * s[j]
        return carry

    lax.fori_loop(0, _CHUNK // 16, mul_body, 0)
    pltpu.sync_copy(rows_v, out_hbm.at[pl.ds(base, _CHUNK)])


def _sc_lookup(x_flat, w_rm, mask_flat):
    mesh = plsc.VectorSubcoreMesh(core_axis_name="c", subcore_axis_name="s")
    return pl.kernel(
        _sc_body,
        out_type=jax.ShapeDtypeStruct((_N, _D), jnp.float32),
        mesh=mesh,
        scratch_types=[
            pltpu.VMEM((_CHUNK,), jnp.int32),
            pltpu.VMEM((_CHUNK, _D), jnp.float32),
            pltpu.VMEM((_CHUNK,), jnp.float32),
            pltpu.VMEM((_CHUNK,), jnp.float32),
            pltpu.SemaphoreType.DMA,
            pltpu.SemaphoreType.DMA,
        ],
        compiler_params=pltpu.CompilerParams(use_tc_tiling_on_sc=False),
    )(x_flat, w_rm, mask_flat)


@jax.jit
def _run(x, weight, mask):
    w_rm = _transpose_tc(weight.T)
    x_flat = x.reshape(-1).astype(jnp.int32)
    mask_flat = mask.reshape(-1)
    out = _sc_lookup(x_flat, w_rm, mask_flat)
    return out.reshape(_B, _F, _D)


def kernel(x, weight, mask):
    return _run(x, weight, mask)


# speedup vs baseline: 2.1350x; 2.1350x over previous
import functools

import jax
import jax.numpy as jnp
from jax import lax
from jax.experimental import pallas as pl
from jax.experimental.pallas import tpu as pltpu
from jax.experimental.pallas import tpu_sc as plsc

_B = 4096
_F = 26
_D = 16
_N = _B * _F
_NW = 32
_CHUNK = _N // _NW
_V = 1000000
_TAU = 0.1
_SIG_HALF = 1.0 / (1.0 + 2.718281828459045 ** (-0.5))

_S = 32768
_C = _S // 8
_GRID = (_V + _S - 1) // _S
_VP = _GRID * _S


def _tr_body(wt_ref, out_ref):
    j = pl.program_id(0)
    col0 = j * _S
    acc = None
    for k in range(8):
        c = wt_ref[:, k * _C:(k + 1) * _C]
        gcol = col0 + k * _C + lax.broadcasted_iota(jnp.int32, (_D, _C), 1)
        c = jnp.where(gcol < _V, c, 0.0)
        ek = jnp.eye(_D, 128, 16 * k, dtype=jnp.float32)
        t = lax.dot_general(c, ek, (((0,), (0,)), ((), ())),
                            preferred_element_type=jnp.float32)
        acc = t if acc is None else acc + t
    out_ref[...] = acc


def _permute_tc(wt):
    out = pl.pallas_call(
        _tr_body,
        grid=(_GRID,),
        in_specs=[pl.BlockSpec((_D, _S), lambda j: (0, j))],
        out_specs=pl.BlockSpec((_S // 8, 128), lambda j: (j, 0)),
        out_shape=jax.ShapeDtypeStruct((_VP * _D // 128, 128), jnp.float32),
    )(wt)
    return out.reshape(_VP, _D)


def _sc_body(x_hbm, w_hbm, m_hbm, out_hbm, idx_v, idxp_v, rows_v, mask_v,
             scale_v, sem_w, sem_m):
    wid = lax.axis_index("s") * 2 + lax.axis_index("c")
    base = wid * _CHUNK
    pltpu.sync_copy(x_hbm.at[pl.ds(base, _CHUNK)], idx_v)

    def perm_body(g, carry):
        i = idx_v[pl.ds(g * 16, 16)]
        p = (i & ~(_S - 1)) | ((i & (_C - 1)) << 3) | ((i >> 12) & 7)
        idxp_v[pl.ds(g * 16, 16)] = p
        return carry

    lax.fori_loop(0, _CHUNK // 16, perm_body, 0)
    cw = pltpu.async_copy(w_hbm.at[idxp_v], rows_v, sem_w)
    cm = pltpu.async_copy(m_hbm.at[idx_v], mask_v, sem_m)
    cm.wait()

    inv_tau = jnp.float32(1.0 / _TAU)
    scale_c = jnp.float32(1.0 / _SIG_HALF)

    def scale_body(g, carry):
        m = mask_v[pl.ds(g * 16, 16)]
        s = scale_c / (1.0 + jnp.exp(m * -inv_tau))
        scale_v[pl.ds(g * 16, 16)] = s
        return carry

    lax.fori_loop(0, _CHUNK // 16, scale_body, 0)
    cw.wait()

    def mul_body(g, carry):
        s = scale_v[pl.ds(g * 16, 16)]
        for j in range(16):
            rows_v[g * 16 + j, :] = rows_v[g * 16 + j, :] * s[j]
        return carry

    lax.fori_loop(0, _CHUNK // 16, mul_body, 0)
    pltpu.sync_copy(rows_v, out_hbm.at[pl.ds(base, _CHUNK)])


def _sc_lookup(x_flat, w_perm, mask_flat):
    mesh = plsc.VectorSubcoreMesh(core_axis_name="c", subcore_axis_name="s")
    return pl.kernel(
        _sc_body,
        out_type=jax.ShapeDtypeStruct((_N, _D), jnp.float32),
        mesh=mesh,
        scratch_types=[
            pltpu.VMEM((_CHUNK,), jnp.int32),
            pltpu.VMEM((_CHUNK,), jnp.int32),
            pltpu.VMEM((_CHUNK, _D), jnp.float32),
            pltpu.VMEM((_CHUNK,), jnp.float32),
            pltpu.VMEM((_CHUNK,), jnp.float32),
            pltpu.SemaphoreType.DMA,
            pltpu.SemaphoreType.DMA,
        ],
        compiler_params=pltpu.CompilerParams(use_tc_tiling_on_sc=False),
    )(x_flat, w_perm, mask_flat)


@jax.jit
def _run(x, weight, mask):
    w_perm = _permute_tc(weight.T)
    x_flat = x.reshape(-1).astype(jnp.int32)
    mask_flat = mask.reshape(-1)
    out = _sc_lookup(x_flat, w_perm, mask_flat)
    return out.reshape(_B, _F, _D)


def kernel(x, weight, mask):
    return _run(x, weight, mask)


# speedup vs baseline: 2.4994x; 1.1706x over previous
import functools

import jax
import jax.numpy as jnp
from jax import lax
from jax.experimental import pallas as pl
from jax.experimental.pallas import tpu as pltpu
from jax.experimental.pallas import tpu_sc as plsc

_B = 4096
_F = 26
_D = 16
_N = _B * _F
_NW = 32
_CHUNK = _N // _NW
_V = 1000000
_TAU = 0.1
_SIG_HALF = 1.0 / (1.0 + 2.718281828459045 ** (-0.5))

_S = 32768
_C = _S // 8
_GRID = (_V + _S - 1) // _S
_VP = _GRID * _S


def _tr_body(wt_ref, sm_ref, out_ref):
    j = pl.program_id(0)
    col0 = j * _S
    sm = sm_ref[...]
    scale = jnp.float32(1.0 / _SIG_HALF) / (
        1.0 + jnp.exp(sm * jnp.float32(-1.0 / _TAU)))
    w = wt_ref[...] * scale
    acc = None
    for k in range(8):
        c = w[:, k * _C:(k + 1) * _C]
        gcol = col0 + k * _C + lax.broadcasted_iota(jnp.int32, (_D, _C), 1)
        c = jnp.where(gcol < _V, c, 0.0)
        ek = jnp.eye(_D, 128, 16 * k, dtype=jnp.float32)
        t = lax.dot_general(c, ek, (((0,), (0,)), ((), ())),
                            preferred_element_type=jnp.float32)
        acc = t if acc is None else acc + t
    out_ref[...] = acc


def _permute_tc(wt, smt):
    out = pl.pallas_call(
        _tr_body,
        grid=(_GRID,),
        in_specs=[pl.BlockSpec((_D, _S), lambda j: (0, j)),
                  pl.BlockSpec((1, _S), lambda j: (0, j))],
        out_specs=pl.BlockSpec((_S // 8, 128), lambda j: (j, 0)),
        out_shape=jax.ShapeDtypeStruct((_VP * _D // 128, 128), jnp.float32),
    )(wt, smt)
    return out.reshape(_VP, _D)


def _sc_body(x_hbm, w_hbm, out_hbm, idx_v, idxp_v, rows_v, sem_w):
    wid = lax.axis_index("s") * 2 + lax.axis_index("c")
    base = wid * _CHUNK
    pltpu.sync_copy(x_hbm.at[pl.ds(base, _CHUNK)], idx_v)

    def perm_body(g, carry):
        i = idx_v[pl.ds(g * 16, 16)]
        p = (i & ~(_S - 1)) | ((i & (_C - 1)) << 3) | ((i >> 12) & 7)
        idxp_v[pl.ds(g * 16, 16)] = p
        return carry

    lax.fori_loop(0, _CHUNK // 16, perm_body, 0)
    pltpu.async_copy(w_hbm.at[idxp_v], rows_v, sem_w).wait()
    pltpu.sync_copy(rows_v, out_hbm.at[pl.ds(base, _CHUNK)])


def _sc_lookup(x_flat, w_perm):
    mesh = plsc.VectorSubcoreMesh(core_axis_name="c", subcore_axis_name="s")
    return pl.kernel(
        _sc_body,
        out_type=jax.ShapeDtypeStruct((_N, _D), jnp.float32),
        mesh=mesh,
        scratch_types=[
            pltpu.VMEM((_CHUNK,), jnp.int32),
            pltpu.VMEM((_CHUNK,), jnp.int32),
            pltpu.VMEM((_CHUNK, _D), jnp.float32),
            pltpu.SemaphoreType.DMA,
        ],
        compiler_params=pltpu.CompilerParams(use_tc_tiling_on_sc=False),
    )(x_flat, w_perm)


@jax.jit
def _run(x, weight, mask):
    w_perm = _permute_tc(weight.T, mask.T)
    x_flat = x.reshape(-1).astype(jnp.int32)
    out = _sc_lookup(x_flat, w_perm)
    return out.reshape(_B, _F, _D)


def kernel(x, weight, mask):
    return _run(x, weight, mask)


# speedup vs baseline: 5.9712x; 2.3891x over previous
import functools

import jax
import jax.numpy as jnp
from jax import lax
from jax.experimental import pallas as pl
from jax.experimental.pallas import tpu as pltpu
from jax.experimental.pallas import tpu_sc as plsc

_B = 4096
_F = 26
_D = 16
_N = _B * _F
_NW = 32
_CHUNK = _N // _NW
_V = 1000000
_TAU = 0.1
_SIG_HALF = 1.0 / (1.0 + 2.718281828459045 ** (-0.5))

_S = 32768
_C = _S // 8
_GRID = (_V + _S - 1) // _S
_VP = _GRID * _S


def _tr_body(wt_ref, sm_ref, out_ref):
    j = pl.program_id(0)
    col0 = j * _S
    sm = sm_ref[...]
    scale = jnp.float32(1.0 / _SIG_HALF) / (
        1.0 + jnp.exp(sm * jnp.float32(-1.0 / _TAU)))
    w = wt_ref[...] * scale
    gcol = col0 + lax.broadcasted_iota(jnp.int32, (1, _S), 1)
    w = jnp.where(gcol < _V, w, 0.0)
    lhs = jnp.concatenate([w[:, k * _C:(k + 1) * _C] for k in range(8)],
                          axis=0)
    out_ref[...] = lax.dot_general(
        lhs, jnp.eye(128, dtype=jnp.float32), (((0,), (0,)), ((), ())),
        preferred_element_type=jnp.float32)


def _permute_tc(wt, smt):
    out = pl.pallas_call(
        _tr_body,
        grid=(_GRID,),
        in_specs=[pl.BlockSpec((_D, _S), lambda j: (0, j)),
                  pl.BlockSpec((1, _S), lambda j: (0, j))],
        out_specs=pl.BlockSpec((_S // 8, 128), lambda j: (j, 0)),
        out_shape=jax.ShapeDtypeStruct((_VP * _D // 128, 128), jnp.float32),
    )(wt, smt)
    return out.reshape(_VP, _D)


def _sc_body(x_hbm, w_hbm, out_hbm, idx_v, idxp_v, rows_v, p5_v, sem_w):
    wid = lax.axis_index("s") * 2 + lax.axis_index("c")
    base = wid * _CHUNK
    pltpu.sync_copy(x_hbm.at[pl.ds(base, _CHUNK)], idx_v)

    def perm_body(g, carry):
        i = idx_v[pl.ds(g * 16, 16)]
        p = (i & ~(_S - 1)) | ((i & (_C - 1)) << 3) | ((i >> 12) & 7)
        idxp_v[pl.ds(g * 16, 16)] = p
        return carry

    lax.fori_loop(0, _CHUNK // 16, perm_body, 0)
    pltpu.async_copy(w_hbm.at[idxp_v], rows_v, sem_w).wait()

    iota26 = lax.broadcasted_iota(jnp.int32, (16,), 0) * _F

    def f_body(f, carry):
        row0 = iota26 + f
        for lg in range(8):
            rows16 = row0 + (16 * _F) * lg
            for d in range(16):
                col = jnp.full((16,), d, jnp.int32)
                vals = plsc.load_gather(rows_v, [rows16, col])
                p5_v[f, d // 8, 0, d % 8, pl.ds(16 * lg, 16)] = vals
        return carry

    lax.fori_loop(0, _F, f_body, 0)
    pltpu.sync_copy(p5_v, out_hbm.at[:, :, pl.ds(wid, 1)])


def _sc_lookup(x_flat, w_perm):
    mesh = plsc.VectorSubcoreMesh(core_axis_name="c", subcore_axis_name="s")
    return pl.kernel(
        _sc_body,
        out_type=jax.ShapeDtypeStruct((_F, 2, _NW, 8, 128), jnp.float32),
        mesh=mesh,
        scratch_types=[
            pltpu.VMEM((_CHUNK,), jnp.int32),
            pltpu.VMEM((_CHUNK,), jnp.int32),
            pltpu.VMEM((_CHUNK, _D), jnp.float32),
            pltpu.VMEM((_F, 2, 1, 8, 128), jnp.float32),
            pltpu.SemaphoreType.DMA,
        ],
        compiler_params=pltpu.CompilerParams(
            use_tc_tiling_on_sc=False, needs_layout_passes=False),
    )(x_flat, w_perm)


@jax.jit
def _run(x, weight, mask):
    w_perm = _permute_tc(weight.T, mask.T)
    x_flat = x.reshape(-1).astype(jnp.int32)
    out5 = _sc_lookup(x_flat, w_perm)
    out = out5.transpose(2, 4, 0, 1, 3).reshape(_B, _F, _D)
    return out


def kernel(x, weight, mask):
    return _run(x, weight, mask)


# speedup vs baseline: 7.5345x; 1.2618x over previous
import functools

import jax
import jax.numpy as jnp
from jax import lax
from jax.experimental import pallas as pl
from jax.experimental.pallas import tpu as pltpu
from jax.experimental.pallas import tpu_sc as plsc

_B = 4096
_F = 26
_D = 16
_N = _B * _F
_NW = 32
_CHUNK = _N // _NW
_V = 1000000
_TAU = 0.1
_SIG_HALF = 1.0 / (1.0 + 2.718281828459045 ** (-0.5))

_S = 32768
_C = _S // 8
_GRID = (_V + _S - 1) // _S
_VP = _GRID * _S


def _tr_body(wt_ref, sm_ref, out_ref):
    j = pl.program_id(0)
    col0 = j * _S
    sm = sm_ref[...]
    scale = jnp.float32(1.0 / _SIG_HALF) / (
        1.0 + jnp.exp(sm * jnp.float32(-1.0 / _TAU)))
    w = wt_ref[...] * scale
    gcol = col0 + lax.broadcasted_iota(jnp.int32, (1, _S), 1)
    w = jnp.where(gcol < _V, w, 0.0)
    lhs = jnp.concatenate([w[:, k * _C:(k + 1) * _C] for k in range(8)],
                          axis=0)
    out_ref[...] = lax.dot_general(
        lhs, jnp.eye(128, dtype=jnp.float32), (((0,), (0,)), ((), ())),
        preferred_element_type=jnp.float32)


def _permute_tc(wt, smt):
    out = pl.pallas_call(
        _tr_body,
        grid=(_GRID,),
        in_specs=[pl.BlockSpec((_D, _S), lambda j: (0, j)),
                  pl.BlockSpec((1, _S), lambda j: (0, j))],
        out_specs=pl.BlockSpec((_S // 8, 128), lambda j: (j, 0)),
        out_shape=jax.ShapeDtypeStruct((_VP * _D // 128, 128), jnp.float32),
    )(wt, smt)
    return out.reshape(_VP, _D)


def _sc_body(x_hbm, w_hbm, out_hbm, idx_v, idxp_v, rows_v, p5_v, sem_w):
    wid = lax.axis_index("s") * 2 + lax.axis_index("c")
    base = wid * _CHUNK
    pltpu.sync_copy(x_hbm.at[pl.ds(base, _CHUNK)], idx_v)

    def perm_body(g, carry):
        i = idx_v[pl.ds(g * 16, 16)]
        p = (i & ~(_S - 1)) | ((i & (_C - 1)) << 3) | ((i >> 12) & 7)
        idxp_v[pl.ds(g * 16, 16)] = p
        return carry

    lax.fori_loop(0, _CHUNK // 16, perm_body, 0)
    pltpu.async_copy(w_hbm.at[idxp_v], rows_v, sem_w).wait()

    d_iota = lax.broadcasted_iota(jnp.int32, (16,), 0)
    ts_vec = d_iota >> 3
    s_vec = d_iota & 7
    zero_vec = jnp.zeros((16,), jnp.int32)

    def l_body(l, n):
        l_vec = jnp.full((16,), 1, jnp.int32) * l

        def f_body(f, n):
            vals = rows_v[n, :]
            f_vec = jnp.full((16,), 1, jnp.int32) * f
            plsc.store_scatter(p5_v, [f_vec, ts_vec, zero_vec, s_vec, l_vec],
                               vals)
            return n + 1

        return lax.fori_loop(0, _F, f_body, n)

    lax.fori_loop(0, 128, l_body, 0)
    pltpu.sync_copy(p5_v.at[:, :, :, :, pl.ds(0, 128)],
                    out_hbm.at[:, :, pl.ds(wid, 1)])


def _sc_lookup(x_flat, w_perm):
    mesh = plsc.VectorSubcoreMesh(core_axis_name="c", subcore_axis_name="s")
    return pl.kernel(
        _sc_body,
        out_type=jax.ShapeDtypeStruct((_F, 2, _NW, 8, 128), jnp.float32),
        mesh=mesh,
        scratch_types=[
            pltpu.VMEM((_CHUNK,), jnp.int32),
            pltpu.VMEM((_CHUNK,), jnp.int32),
            pltpu.VMEM((_CHUNK, _D), jnp.float32),
            pltpu.VMEM((_F, 2, 1, 8, 129), jnp.float32),
            pltpu.SemaphoreType.DMA,
        ],
        compiler_params=pltpu.CompilerParams(
            use_tc_tiling_on_sc=False, needs_layout_passes=False),
    )(x_flat, w_perm)


@jax.jit
def _run(x, weight, mask):
    w_perm = _permute_tc(weight.T, mask.T)
    x_flat = x.reshape(-1).astype(jnp.int32)
    out5 = _sc_lookup(x_flat, w_perm)
    out = out5.transpose(2, 4, 0, 1, 3).reshape(_B, _F, _D)
    return out


def kernel(x, weight, mask):
    return _run(x, weight, mask)


# speedup vs baseline: 8.4493x; 1.1214x over previous
import functools

import jax
import jax.numpy as jnp
from jax import lax
from jax.experimental import pallas as pl
from jax.experimental.pallas import tpu as pltpu
from jax.experimental.pallas import tpu_sc as plsc

_B = 4096
_F = 26
_D = 16
_N = _B * _F
_NW = 32
_CHUNK = _N // _NW
_V = 1000000
_TAU = 0.1
_SIG_HALF = 1.0 / (1.0 + 2.718281828459045 ** (-0.5))

_S = 65536
_C = _S // 8
_LC = _C.bit_length() - 1
_GRID = (_V + _S - 1) // _S
_VP = _GRID * _S


def _tr_body(wt_ref, sm_ref, out_ref):
    j = pl.program_id(0)
    col0 = j * _S
    sm = sm_ref[...]
    scale = jnp.float32(1.0 / _SIG_HALF) / (
        1.0 + jnp.exp(sm * jnp.float32(-1.0 / _TAU)))
    w = wt_ref[...] * scale
    gcol = col0 + lax.broadcasted_iota(jnp.int32, (1, _S), 1)
    w = jnp.where(gcol < _V, w, 0.0)
    lhs = jnp.concatenate([w[:, k * _C:(k + 1) * _C] for k in range(8)],
                          axis=0)
    out_ref[...] = lax.dot_general(
        lhs, jnp.eye(128, dtype=jnp.float32), (((0,), (0,)), ((), ())),
        preferred_element_type=jnp.float32)


def _permute_tc(wt, smt):
    out = pl.pallas_call(
        _tr_body,
        grid=(_GRID,),
        in_specs=[pl.BlockSpec((_D, _S), lambda j: (0, j)),
                  pl.BlockSpec((1, _S), lambda j: (0, j))],
        out_specs=pl.BlockSpec((_S // 8, 128), lambda j: (j, 0)),
        out_shape=jax.ShapeDtypeStruct((_VP * _D // 128, 128), jnp.float32),
    )(wt, smt)
    return out.reshape(_VP, _D)


def _sc_body(x_hbm, w_hbm, out_hbm, idx_v, idxp_v, rows_v, p5_v, sem_w):
    wid = lax.axis_index("s") * 2 + lax.axis_index("c")
    base = wid * _CHUNK
    pltpu.sync_copy(x_hbm.at[pl.ds(base, _CHUNK)], idx_v)

    def perm_body(g, carry):
        i = idx_v[pl.ds(g * 16, 16)]
        p = (i & ~(_S - 1)) | ((i & (_C - 1)) << 3) | ((i >> _LC) & 7)
        idxp_v[pl.ds(g * 16, 16)] = p
        return carry

    lax.fori_loop(0, _CHUNK // 16, perm_body, 0)
    pltpu.async_copy(w_hbm.at[idxp_v], rows_v, sem_w).wait()

    d_iota = lax.broadcasted_iota(jnp.int32, (16,), 0)
    ts_vec = d_iota >> 3
    s_vec = d_iota & 7
    zero_vec = jnp.zeros((16,), jnp.int32)
    one_vec = jnp.ones((16,), jnp.int32)

    def f_body(f, f_vec):
        def l_body(l, carry):
            n, l_vec = carry
            vals = rows_v[n, :]
            plsc.store_scatter(p5_v, [f_vec, ts_vec, zero_vec, s_vec, l_vec],
                               vals)
            return n + _F, l_vec + one_vec

        lax.fori_loop(0, 128, l_body, (f, zero_vec), unroll=4)
        return f_vec + one_vec

    lax.fori_loop(0, _F, f_body, zero_vec)
    pltpu.sync_copy(p5_v.at[:, :, :, :, pl.ds(0, 128)],
                    out_hbm.at[:, :, pl.ds(wid, 1)])


def _sc_lookup(x_flat, w_perm):
    mesh = plsc.VectorSubcoreMesh(core_axis_name="c", subcore_axis_name="s")
    return pl.kernel(
        _sc_body,
        out_type=jax.ShapeDtypeStruct((_F, 2, _NW, 8, 128), jnp.float32),
        mesh=mesh,
        scratch_types=[
            pltpu.VMEM((_CHUNK,), jnp.int32),
            pltpu.VMEM((_CHUNK,), jnp.int32),
            pltpu.VMEM((_CHUNK, _D), jnp.float32),
            pltpu.VMEM((_F, 2, 1, 8, 129), jnp.float32),
            pltpu.SemaphoreType.DMA,
        ],
        compiler_params=pltpu.CompilerParams(
            use_tc_tiling_on_sc=False, needs_layout_passes=False),
    )(x_flat, w_perm)


@jax.jit
def _run(x, weight, mask):
    w_perm = _permute_tc(weight.T, mask.T)
    x_flat = x.reshape(-1).astype(jnp.int32)
    out5 = _sc_lookup(x_flat, w_perm)
    out = out5.transpose(2, 4, 0, 1, 3).reshape(_B, _F, _D)
    return out


def kernel(x, weight, mask):
    return _run(x, weight, mask)


# speedup vs baseline: 8.5844x; 1.0160x over previous
import functools

import jax
import jax.numpy as jnp
from jax import lax
from jax.experimental import pallas as pl
from jax.experimental.pallas import tpu as pltpu
from jax.experimental.pallas import tpu_sc as plsc

_B = 4096
_F = 26
_D = 16
_N = _B * _F
_NW = 32
_CHUNK = _N // _NW
_V = 1000000
_TAU = 0.1
_SIG_HALF = 1.0 / (1.0 + 2.718281828459045 ** (-0.5))

_S = 65536
_C = _S // 8
_LC = _C.bit_length() - 1
_GRID = (_V + _S - 1) // _S
_VP = _GRID * _S


def _tr_body(wt_ref, sm_ref, out_ref):
    j = pl.program_id(0)
    col0 = j * _S
    sm = sm_ref[...]
    scale = jnp.float32(1.0 / _SIG_HALF) / (
        1.0 + jnp.exp(sm * jnp.float32(-1.0 / _TAU)))
    w = wt_ref[...] * scale
    gcol = col0 + lax.broadcasted_iota(jnp.int32, (1, _S), 1)
    w = jnp.where(gcol < _V, w, 0.0)
    lhs = jnp.concatenate([w[:, k * _C:(k + 1) * _C] for k in range(8)],
                          axis=0)
    out_ref[...] = lax.dot_general(
        lhs, jnp.eye(128, dtype=jnp.float32), (((0,), (0,)), ((), ())),
        preferred_element_type=jnp.float32)


def _permute_tc(wt, smt):
    out = pl.pallas_call(
        _tr_body,
        grid=(_GRID,),
        in_specs=[pl.BlockSpec((_D, _S), lambda j: (0, j)),
                  pl.BlockSpec((1, _S), lambda j: (0, j))],
        out_specs=pl.BlockSpec((_S // 8, 128), lambda j: (j, 0)),
        out_shape=jax.ShapeDtypeStruct((_VP * _D // 128, 128), jnp.float32),
    )(wt, smt)
    return out.reshape(_VP, _D)


_H = _CHUNK // 2


def _sc_body(x_hbm, w_hbm, out_hbm, idx_v, idxp_a, idxp_b, rows_a, rows_b,
             p5_v, sem_a, sem_b):
    wid = lax.axis_index("s") * 2 + lax.axis_index("c")
    base = wid * _CHUNK
    pltpu.sync_copy(x_hbm.at[pl.ds(base, _CHUNK)], idx_v)

    def perm_body(off, dst):
        def body(g, carry):
            i = idx_v[pl.ds(off + g * 16, 16)]
            p = (i & ~(_S - 1)) | ((i & (_C - 1)) << 3) | ((i >> _LC) & 7)
            dst[pl.ds(g * 16, 16)] = p
            return carry
        return body

    lax.fori_loop(0, _H // 16, perm_body(0, idxp_a), 0)
    cwa = pltpu.async_copy(w_hbm.at[idxp_a], rows_a, sem_a)
    lax.fori_loop(0, _H // 16, perm_body(_H, idxp_b), 0)
    cwb = pltpu.async_copy(w_hbm.at[idxp_b], rows_b, sem_b)

    d_iota = lax.broadcasted_iota(jnp.int32, (16,), 0)
    ts_vec = d_iota >> 3
    s_vec = d_iota & 7
    zero_vec = jnp.zeros((16,), jnp.int32)
    one_vec = jnp.ones((16,), jnp.int32)

    def transpose_half(rows_ref, l0):
        l0_vec = jnp.full((16,), l0, jnp.int32)

        def f_body(f, f_vec):
            def l_body(l, carry):
                n, l_vec = carry
                vals = rows_ref[n, :]
                plsc.store_scatter(
                    p5_v, [f_vec, ts_vec, zero_vec, s_vec, l_vec], vals)
                return n + _F, l_vec + one_vec

            lax.fori_loop(0, 64, l_body, (f, l0_vec), unroll=8)
            return f_vec + one_vec

        lax.fori_loop(0, _F, f_body, zero_vec)

    cwa.wait()
    transpose_half(rows_a, 0)
    cwb.wait()
    transpose_half(rows_b, 64)
    pltpu.sync_copy(p5_v.at[:, :, :, :, pl.ds(0, 128)],
                    out_hbm.at[:, :, pl.ds(wid, 1)])


def _sc_lookup(x_flat, w_perm):
    mesh = plsc.VectorSubcoreMesh(core_axis_name="c", subcore_axis_name="s")
    return pl.kernel(
        _sc_body,
        out_type=jax.ShapeDtypeStruct((_F, 2, _NW, 8, 128), jnp.float32),
        mesh=mesh,
        scratch_types=[
            pltpu.VMEM((_CHUNK,), jnp.int32),
            pltpu.VMEM((_H,), jnp.int32),
            pltpu.VMEM((_H,), jnp.int32),
            pltpu.VMEM((_H, _D), jnp.float32),
            pltpu.VMEM((_H, _D), jnp.float32),
            pltpu.VMEM((_F, 2, 1, 8, 129), jnp.float32),
            pltpu.SemaphoreType.DMA,
            pltpu.SemaphoreType.DMA,
        ],
        compiler_params=pltpu.CompilerParams(
            use_tc_tiling_on_sc=False, needs_layout_passes=False),
    )(x_flat, w_perm)


@jax.jit
def _run(x, weight, mask):
    w_perm = _permute_tc(weight.T, mask.T)
    x_flat = x.reshape(-1).astype(jnp.int32)
    out5 = _sc_lookup(x_flat, w_perm)
    out = out5.transpose(2, 4, 0, 1, 3).reshape(_B, _F, _D)
    return out


def kernel(x, weight, mask):
    return _run(x, weight, mask)


# speedup vs baseline: 8.6091x; 1.0029x over previous
import functools

import jax
import jax.numpy as jnp
from jax import lax
from jax.experimental import pallas as pl
from jax.experimental.pallas import tpu as pltpu
from jax.experimental.pallas import tpu_sc as plsc

_B = 4096
_F = 26
_D = 16
_N = _B * _F
_NW = 32
_CHUNK = _N // _NW
_V = 1000000
_TAU = 0.1
_SIG_HALF = 1.0 / (1.0 + 2.718281828459045 ** (-0.5))

_S = 65536
_C = _S // 8
_LC = _C.bit_length() - 1
_GRID = (_V + _S - 1) // _S
_VP = _GRID * _S


def _tr_body(wt_ref, sm_ref, out_ref):
    j = pl.program_id(0)
    col0 = j * _S
    sm = sm_ref[...]
    scale = jnp.float32(1.0 / _SIG_HALF) / (
        1.0 + jnp.exp(sm * jnp.float32(-1.0 / _TAU)))
    w = wt_ref[...] * scale
    gcol = col0 + lax.broadcasted_iota(jnp.int32, (1, _S), 1)
    w = jnp.where(gcol < _V, w, 0.0)
    lhs = jnp.concatenate([w[:, k * _C:(k + 1) * _C] for k in range(8)],
                          axis=0)
    out_ref[...] = lax.dot_general(
        lhs, jnp.eye(128, dtype=jnp.float32), (((0,), (0,)), ((), ())),
        preferred_element_type=jnp.float32)


def _permute_tc(wt, smt):
    out = pl.pallas_call(
        _tr_body,
        grid=(_GRID,),
        in_specs=[pl.BlockSpec((_D, _S), lambda j: (0, j)),
                  pl.BlockSpec((1, _S), lambda j: (0, j))],
        out_specs=pl.BlockSpec((_S // 8, 128), lambda j: (j, 0)),
        out_shape=jax.ShapeDtypeStruct((_VP * _D // 128, 128), jnp.float32),
    )(wt, smt)
    return out.reshape(_VP, _D)


_NPH = 4
_H = _CHUNK // _NPH
_LH = 128 // _NPH


def _sc_body(x_hbm, w_hbm, out_hbm, idx_v, idxp, rows, p5_v, sems):
    wid = lax.axis_index("s") * 2 + lax.axis_index("c")
    base = wid * _CHUNK
    pltpu.sync_copy(x_hbm.at[pl.ds(base, _CHUNK)], idx_v)

    def perm_body(off, dst):
        def body(g, carry):
            i = idx_v[pl.ds(off + g * 16, 16)]
            p = (i & ~(_S - 1)) | ((i & (_C - 1)) << 3) | ((i >> _LC) & 7)
            dst[pl.ds(g * 16, 16)] = p
            return carry
        return body

    copies = []
    for ph in range(_NPH):
        lax.fori_loop(0, _H // 16, perm_body(ph * _H, idxp[ph]), 0)
        copies.append(pltpu.async_copy(w_hbm.at[idxp[ph]], rows[ph],
                                       sems[ph]))

    d_iota = lax.broadcasted_iota(jnp.int32, (16,), 0)
    ts_vec = d_iota >> 3
    s_vec = d_iota & 7
    zero_vec = jnp.zeros((16,), jnp.int32)
    one_vec = jnp.ones((16,), jnp.int32)

    def transpose_phase(rows_ref, l0):
        l0_vec = jnp.full((16,), l0, jnp.int32)

        def f_body(f, f_vec):
            def l_body(l, carry):
                n, l_vec = carry
                vals = rows_ref[n, :]
                plsc.store_scatter(
                    p5_v, [f_vec, ts_vec, zero_vec, s_vec, l_vec], vals)
                return n + _F, l_vec + one_vec

            lax.fori_loop(0, _LH, l_body, (f, l0_vec), unroll=8)
            return f_vec + one_vec

        lax.fori_loop(0, _F, f_body, zero_vec)

    for ph in range(_NPH):
        copies[ph].wait()
        transpose_phase(rows[ph], ph * _LH)
    pltpu.sync_copy(p5_v.at[:, :, :, :, pl.ds(0, 128)],
                    out_hbm.at[:, :, pl.ds(wid, 1)])


def _sc_lookup(x_flat, w_perm):
    mesh = plsc.VectorSubcoreMesh(core_axis_name="c", subcore_axis_name="s")
    return pl.kernel(
        _sc_body,
        out_type=jax.ShapeDtypeStruct((_F, 2, _NW, 8, 128), jnp.float32),
        mesh=mesh,
        scratch_types=[
            pltpu.VMEM((_CHUNK,), jnp.int32),
            [pltpu.VMEM((_H,), jnp.int32) for _ in range(_NPH)],
            [pltpu.VMEM((_H, _D), jnp.float32) for _ in range(_NPH)],
            pltpu.VMEM((_F, 2, 1, 8, 129), jnp.float32),
            [pltpu.SemaphoreType.DMA for _ in range(_NPH)],
        ],
        compiler_params=pltpu.CompilerParams(
            use_tc_tiling_on_sc=False, needs_layout_passes=False),
    )(x_flat, w_perm)


@jax.jit
def _run(x, weight, mask):
    w_perm = _permute_tc(weight.T, mask.T)
    x_flat = x.reshape(-1).astype(jnp.int32)
    out5 = _sc_lookup(x_flat, w_perm)
    out = out5.transpose(2, 4, 0, 1, 3).reshape(_B, _F, _D)
    return out


def kernel(x, weight, mask):
    return _run(x, weight, mask)


# speedup vs baseline: 9.4686x; 1.0998x over previous
import functools

import jax
import jax.numpy as jnp
from jax import lax
from jax.experimental import pallas as pl
from jax.experimental.pallas import tpu as pltpu
from jax.experimental.pallas import tpu_sc as plsc

_B = 4096
_F = 26
_D = 16
_N = _B * _F
_NW = 32
_CHUNK = _N // _NW
_V = 1000000
_TAU = 0.1
_SIG_HALF = 1.0 / (1.0 + 2.718281828459045 ** (-0.5))

_S = 65536
_C = _S // 8
_LC = _C.bit_length() - 1
_GRID = (_V + _S - 1) // _S
_VP = _GRID * _S


def _tr_body(wt_ref, sm_ref, out_ref):
    j = pl.program_id(0)
    col0 = j * _S
    sm = sm_ref[...]
    scale = jnp.float32(1.0 / _SIG_HALF) / (
        1.0 + jnp.exp(sm * jnp.float32(-1.0 / _TAU)))
    w = wt_ref[...] * scale
    gcol = col0 + lax.broadcasted_iota(jnp.int32, (1, _S), 1)
    w = jnp.where(gcol < _V, w, 0.0)
    lhs = jnp.concatenate([w[:, k * _C:(k + 1) * _C] for k in range(8)],
                          axis=0)
    out_ref[...] = lax.dot_general(
        lhs, jnp.eye(128, dtype=jnp.float32), (((0,), (0,)), ((), ())),
        preferred_element_type=jnp.float32)


def _permute_tc(wt, smt):
    out = pl.pallas_call(
        _tr_body,
        grid=(_GRID,),
        in_specs=[pl.BlockSpec((_D, _S), lambda j: (0, j)),
                  pl.BlockSpec((1, _S), lambda j: (0, j))],
        out_specs=pl.BlockSpec((_S // 8, 128), lambda j: (j, 0)),
        out_shape=jax.ShapeDtypeStruct((_VP * _D // 128, 128), jnp.float32),
    )(wt, smt)
    return out.reshape(_VP, _D)


_NPH = 4
_H = _CHUNK // _NPH
_LH = 128 // _NPH


def _sc_body(x_hbm, w_hbm, out_hbm, idx_v, idxp, rows, p5_v, sems):
    wid = lax.axis_index("s") * 2 + lax.axis_index("c")
    base = wid * _CHUNK
    pltpu.sync_copy(x_hbm.at[pl.ds(base, _CHUNK)], idx_v)

    def perm_body(off, dst):
        def body(g, carry):
            i = idx_v[pl.ds(off + g * 16, 16)]
            p = (i & ~(_S - 1)) | ((i & (_C - 1)) << 3) | ((i >> _LC) & 7)
            dst[pl.ds(g * 16, 16)] = p
            return carry
        return body

    copies = []
    for ph in range(_NPH):
        lax.fori_loop(0, _H // 16, perm_body(ph * _H, idxp[ph]), 0)
        copies.append(pltpu.async_copy(w_hbm.at[idxp[ph]], rows[ph],
                                       sems[ph]))

    d_iota = lax.broadcasted_iota(jnp.int32, (16,), 0)
    ts_vec = d_iota >> 3
    s_vec = d_iota & 7
    zero_vec = jnp.zeros((16,), jnp.int32)
    one_vec = jnp.ones((16,), jnp.int32)

    def transpose_phase(rows_ref, l0):
        l0_vec = jnp.full((16,), l0, jnp.int32)

        def f_body(f, f_vec):
            @plsc.parallel_loop(0, _LH, unroll=8)
            def l_body(l):
                vals = rows_ref[l * _F + f, :]
                l_vec = l0_vec + l
                plsc.store_scatter(
                    p5_v, [f_vec, ts_vec, zero_vec, s_vec, l_vec], vals)

            return f_vec + one_vec

        lax.fori_loop(0, _F, f_body, zero_vec)

    for ph in range(_NPH):
        copies[ph].wait()
        transpose_phase(rows[ph], ph * _LH)
    pltpu.sync_copy(p5_v.at[:, :, :, :, pl.ds(0, 128)],
                    out_hbm.at[:, :, pl.ds(wid, 1)])


def _sc_lookup(x_flat, w_perm):
    mesh = plsc.VectorSubcoreMesh(core_axis_name="c", subcore_axis_name="s")
    return pl.kernel(
        _sc_body,
        out_type=jax.ShapeDtypeStruct((_F, 2, _NW, 8, 128), jnp.float32),
        mesh=mesh,
        scratch_types=[
            pltpu.VMEM((_CHUNK,), jnp.int32),
            [pltpu.VMEM((_H,), jnp.int32) for _ in range(_NPH)],
            [pltpu.VMEM((_H, _D), jnp.float32) for _ in range(_NPH)],
            pltpu.VMEM((_F, 2, 1, 8, 129), jnp.float32),
            [pltpu.SemaphoreType.DMA for _ in range(_NPH)],
        ],
        compiler_params=pltpu.CompilerParams(
            use_tc_tiling_on_sc=False, needs_layout_passes=False),
    )(x_flat, w_perm)


@jax.jit
def _run(x, weight, mask):
    w_perm = _permute_tc(weight.T, mask.T)
    x_flat = x.reshape(-1).astype(jnp.int32)
    out5 = _sc_lookup(x_flat, w_perm)
    out = out5.transpose(2, 4, 0, 1, 3).reshape(_B, _F, _D)
    return out


def kernel(x, weight, mask):
    return _run(x, weight, mask)
